# Initial kernel scaffold; baseline (speedup 1.0000x reference)
import jax
import jax.numpy as jnp
from jax.experimental import pallas as pl


def kernel(h_batch, W):
    raise NotImplementedError("write your pallas kernel here")


# speedup vs baseline: 90.5436x; 90.5436x over previous
import jax
import jax.numpy as jnp
from jax.experimental import pallas as pl
from jax.experimental.pallas import tpu as pltpu

_B = 16384
_K = 8192
_D = 64
_TB = 256
_BETA = 0.25


def _normalize_rows(x, eps=1e-12):
    norm = jnp.linalg.norm(x, axis=1, keepdims=True)
    return x / jnp.maximum(norm, eps)


def _vq_body(nh_ref, nw_ref, w_ref, h_ref,
             onehot_ref, zq_ref, idx_ref, loss_ref, perp_ref,
             counts_ref, lacc_ref):
    b = pl.program_id(0)
    nb = pl.num_programs(0)

    nh = nh_ref[...]
    d = jax.lax.dot_general(
        nh, nw_ref[...], (((1,), (1,)), ((), ())),
        preferred_element_type=jnp.float32)
    m = jnp.max(d, axis=1, keepdims=True)
    col = jax.lax.broadcasted_iota(jnp.int32, (_TB, _K), 1)
    sel = jnp.where(d == m, col, -1)
    idx = jnp.max(sel, axis=1, keepdims=True)

    onehot = (col == idx).astype(jnp.float32)
    onehot_ref[...] = onehot
    idx_ref[...] = idx[:, 0]

    zq = jax.lax.dot_general(
        onehot, w_ref[...], (((1,), (0,)), ((), ())),
        preferred_element_type=jnp.float32)
    h = h_ref[...]
    zq_ref[...] = h + (zq - h)

    @pl.when(b == 0)
    def _init():
        lacc_ref[...] = jnp.zeros_like(lacc_ref)
        counts_ref[...] = jnp.zeros_like(counts_ref)
        perp_ref[...] = jnp.zeros_like(perp_ref)

    diff = zq - h
    lacc_ref[...] += jnp.sum(diff * diff).reshape(1, 1)
    counts_ref[...] += jnp.sum(onehot, axis=0, keepdims=True)
    loss_ref[...] = (1.0 + _BETA) * (1.0 / (_B * _D)) * lacc_ref[...]

    @pl.when(b == nb - 1)
    def _fin():
        p = counts_ref[...] * (1.0 / _B)
        ent = jnp.sum(p * jnp.log(p + 1e-10))
        perp_ref[...] = jnp.exp(-ent).reshape(1, 1)


def kernel(h_batch, W):
    nh = _normalize_rows(jax.lax.stop_gradient(h_batch))
    nw = _normalize_rows(W)
    grid = (_B // _TB,)
    onehot, zq, idx, loss, perp = pl.pallas_call(
        _vq_body,
        grid=grid,
        in_specs=[
            pl.BlockSpec((_TB, _D), lambda b: (b, 0)),
            pl.BlockSpec((_K, _D), lambda b: (0, 0)),
            pl.BlockSpec((_K, _D), lambda b: (0, 0)),
            pl.BlockSpec((_TB, _D), lambda b: (b, 0)),
        ],
        out_specs=[
            pl.BlockSpec((_TB, _K), lambda b: (b, 0)),
            pl.BlockSpec((_TB, _D), lambda b: (b, 0)),
            pl.BlockSpec((_TB,), lambda b: (b,)),
            pl.BlockSpec((1, 1), lambda b: (0, 0)),
            pl.BlockSpec((1, 1), lambda b: (0, 0)),
        ],
        out_shape=[
            jax.ShapeDtypeStruct((_B, _K), jnp.float32),
            jax.ShapeDtypeStruct((_B, _D), jnp.float32),
            jax.ShapeDtypeStruct((_B,), jnp.int32),
            jax.ShapeDtypeStruct((1, 1), jnp.float32),
            jax.ShapeDtypeStruct((1, 1), jnp.float32),
        ],
        scratch_shapes=[
            pltpu.VMEM((1, _K), jnp.float32),
            pltpu.VMEM((1, 1), jnp.float32),
        ],
        compiler_params=pltpu.CompilerParams(
            dimension_semantics=("arbitrary",),
        ),
    )(nh, nw, W, h_batch)
    return (zq, loss[0, 0], perp[0, 0], onehot, idx)
